# Initial kernel scaffold; baseline (speedup 1.0000x reference)
import jax
import jax.numpy as jnp
from jax.experimental import pallas as pl


def kernel(token_fields, float_fields, token_seq_field, token_table, float_table, seq_table):
    raise NotImplementedError("write your pallas kernel here")


# speedup vs baseline: 3.6014x; 3.6014x over previous
import functools

import jax
import jax.numpy as jnp
from jax import lax
from jax.experimental import pallas as pl
from jax.experimental.pallas import tpu as pltpu
from jax.experimental.pallas import tpu_sc as plsc

_N_TOK = 8
_N_FLT = 4
_S = 20
_D = 64
_FIELD_DIM = 100000
_NC, _NS, _LANES = 2, 16, 16
_NW = _NC * _NS
_C = 32
_OUT_FIELDS = _N_TOK + _N_FLT + 1


@functools.lru_cache(maxsize=None)
def _build(P):
  per_w = P // _NW
  n_chunks = per_w // _C
  assert per_w * _NW == P and n_chunks * _C == per_w

  mesh = plsc.VectorSubcoreMesh(
      core_axis_name="c", subcore_axis_name="s",
      num_cores=_NC, num_subcores=_NS)

  @functools.partial(
      pl.kernel,
      out_type=jax.ShapeDtypeStruct((P * _OUT_FIELDS, _D), jnp.float32),
      mesh=mesh,
      scratch_types=[
          pltpu.VMEM((_C * _N_TOK,), jnp.int32),
          pltpu.VMEM((_C * _S,), jnp.int32),
          pltpu.VMEM((_C * _N_FLT,), jnp.float32),
          pltpu.VMEM((_N_TOK, _C), jnp.int32),
          pltpu.VMEM((_N_TOK * _C, _D), jnp.float32),
          pltpu.VMEM((_C * _S, _D), jnp.float32),
          pltpu.VMEM((_C, _D), jnp.float32),
          pltpu.VMEM((_N_FLT * _C, _D), jnp.float32),
          pltpu.VMEM((_C,), jnp.float32),
          pltpu.VMEM((_C,), jnp.float32),
          pltpu.VMEM((_N_FLT, _D), jnp.float32),
          pltpu.VMEM((_D,), jnp.float32),
          pltpu.VMEM((3, 128), jnp.int32),
          pltpu.VMEM((1, _C), jnp.int32),
          pltpu.SemaphoreType.DMA,
          pltpu.SemaphoreType.DMA,
          pltpu.SemaphoreType.DMA,
      ],
      compiler_params=pltpu.CompilerParams(
          needs_layout_passes=False, use_tc_tiling_on_sc=False),
  )
  def k(tok_idx, fvals, seq_idx, tok_tab, f_tab, seq_tab, out,
        rawtok_v, seqidx_v, fvals_v, tokt_v, tokbuf_v, sbuf_v, acc_v,
        fbuf_v, inv_v, n0_v, ftab_v, row0_v, widx_v, sidx_v,
        ssem, gsem, wsem):
    wid = lax.axis_index("s") * _NC + lax.axis_index("c")
    iota = lax.iota(jnp.int32, _LANES)
    pltpu.sync_copy(f_tab, ftab_v)
    pltpu.sync_copy(seq_tab.at[0], row0_v)

    def chunk(ci, _):
      base = wid * per_w + ci * _C

      c1 = pltpu.async_copy(tok_idx.at[pl.ds(base * _N_TOK, _C * _N_TOK)],
                            rawtok_v, ssem)
      c2 = pltpu.async_copy(seq_idx.at[pl.ds(base * _S, _C * _S)],
                            seqidx_v, ssem)
      c3 = pltpu.async_copy(fvals.at[pl.ds(base * _N_FLT, _C * _N_FLT)],
                            fvals_v, ssem)
      c1.wait(); c2.wait(); c3.wait()

      for g in range(_C // _LANES):
        p16 = g * _LANES + iota
        dst16 = (base + p16) * _OUT_FIELDS
        for f in range(_N_TOK):
          src = plsc.load_gather(rawtok_v, [p16 * _N_TOK + f])
          tokt_v[f, pl.ds(g * _LANES, _LANES)] = src + f * _FIELD_DIM
          kk = f * _C + g * _LANES
          widx_v[kk // 128, pl.ds(kk % 128, _LANES)] = dst16 + f
        for f in range(_N_FLT):
          kk = f * _C + g * _LANES
          widx_v[2, pl.ds(kk, _LANES)] = dst16 + (_N_TOK + f)
        sidx_v[0, pl.ds(g * _LANES, _LANES)] = dst16 + (_N_TOK + _N_FLT)

      for g in range(_C // _LANES):
        pvec = (g * _LANES + iota) * _S
        cnt = jnp.zeros((_LANES,), jnp.int32)
        for s in range(_S):
          v = plsc.load_gather(seqidx_v, [pvec + s])
          cnt = cnt + jnp.where(v != 0, 1, 0).astype(jnp.int32)
        cntf = cnt.astype(jnp.float32)
        inv = jnp.where(cnt > 0, 1.0 / (cntf + jnp.float32(1e-8)),
                        jnp.float32(0.0))
        inv_v[pl.ds(g * _LANES, _LANES)] = inv
        n0_v[pl.ds(g * _LANES, _LANES)] = jnp.float32(_S) - cntf

      gathers = []
      for f in range(_N_TOK):
        gathers.append(pltpu.async_copy(
            tok_tab.at[tokt_v.at[f]],
            tokbuf_v.at[pl.ds(f * _C, _C)], gsem))
      for g in range(_C * _S // 128):
        gathers.append(pltpu.async_copy(
            seq_tab.at[seqidx_v.at[pl.ds(g * 128, 128)]],
            sbuf_v.at[pl.ds(g * 128, 128)], gsem))

      def fpos(p, carry):
        for f in range(_N_FLT):
          vb = plsc.load_gather(
              fvals_v, [jnp.full((_LANES,), p * _N_FLT + f, jnp.int32)])
          for j in range(_D // _LANES):
            sl = pl.ds(j * _LANES, _LANES)
            fbuf_v[f * _C + p, sl] = vb * ftab_v[f, sl]
        return carry
      lax.fori_loop(0, _C, fpos, None)

      for dsc in gathers:
        dsc.wait()

      writes = []
      for g in range(2):
        writes.append(pltpu.async_copy(
            tokbuf_v.at[pl.ds(g * 128, 128)], out.at[widx_v.at[g]], wsem))
      writes.append(pltpu.async_copy(
          fbuf_v, out.at[widx_v.at[2]], wsem))

      def spos(p, carry):
        pb = jnp.full((_LANES,), p, jnp.int32)
        invb = plsc.load_gather(inv_v, [pb])
        n0b = plsc.load_gather(n0_v, [pb])
        for j in range(_D // _LANES):
          sl = pl.ds(j * _LANES, _LANES)
          vals = [sbuf_v[p * _S + s, sl] for s in range(_S)]
          while len(vals) > 1:
            nxt = [vals[i] + vals[i + 1] for i in range(0, len(vals) - 1, 2)]
            if len(vals) % 2:
              nxt.append(vals[-1])
            vals = nxt
          acc_v[p, sl] = (vals[0] - n0b * row0_v[sl]) * invb
        return carry
      lax.fori_loop(0, _C, spos, None)

      writes.append(pltpu.async_copy(acc_v, out.at[sidx_v.at[0]], wsem))
      for dsc in writes:
        dsc.wait()
      return _

    lax.fori_loop(0, n_chunks, chunk, None)

  return k


def kernel(token_fields, float_fields, token_seq_field,
           token_table, float_table, seq_table):
  B, L, nf = token_fields.shape
  P = B * L
  k = _build(P)
  out = k(token_fields.reshape(P * nf),
          float_fields.reshape(-1),
          token_seq_field.reshape(-1),
          token_table, float_table, seq_table)
  return out.reshape(B, L, _OUT_FIELDS, _D)
